# baseline (speedup 1.0000x reference)
import functools

import jax
import jax.numpy as jnp
from jax import lax
from jax.experimental import pallas as pl
from jax.experimental.pallas import tpu as pltpu
from jax.experimental.pallas import tpu_sc as plsc

EPS = 1e-05
DOWNSAMPLE = 0.5
_NC = 2
_NS = 16
_L = 16


def _tc_body(x_ref, w_ref, b_ref, off_ref, att_ref, dst_ref, flat_ref,
             *, H, W, dh, dw, P):
    j = pl.program_id(1)
    xb = x_ref[0]

    oa = jnp.dot(w_ref[...], xb, preferred_element_type=jnp.float32) + b_ref[...]

    p = j * P + jax.lax.broadcasted_iota(jnp.int32, (1, P), 1)
    hh = (p // W).astype(jnp.float32) * (1.0 / float(H))
    ww = (p % W).astype(jnp.float32) * (1.0 / float(W))

    d0 = jnp.clip(hh + oa[0:1], 0.0, 1.0 - EPS)
    d1 = jnp.clip(ww + oa[1:2], 0.0, 1.0 - EPS)
    dr = jnp.floor(d0 * dh).astype(jnp.int32)
    dc = jnp.floor(d1 * dw).astype(jnp.int32)

    off_ref[0] = oa[0:2]
    att_ref[0] = oa[2:3]
    dst_ref[0] = jnp.concatenate([dr, dc], axis=0)
    flat_ref[0] = dr * dw + dc


def _tc_phase(xf, conv_w, bb, H, W, dh, dw, b0, nb):
    B, C, HW = xf.shape
    P = 25088
    assert HW % P == 0
    NPB = HW // P
    return pl.pallas_call(
        functools.partial(_tc_body, H=H, W=W, dh=dh, dw=dw, P=P),
        grid=(nb, NPB),
        in_specs=[
            pl.BlockSpec((1, C, P), lambda b, j: (b + b0, 0, j)),
            pl.BlockSpec((3, C), lambda b, j: (0, 0)),
            pl.BlockSpec((3, 1), lambda b, j: (0, 0)),
        ],
        out_specs=[
            pl.BlockSpec((1, 2, P), lambda b, j: (b, 0, j)),
            pl.BlockSpec((1, 1, P), lambda b, j: (b, 0, j)),
            pl.BlockSpec((1, 2, P), lambda b, j: (b, 0, j)),
            pl.BlockSpec((1, 1, P), lambda b, j: (b, 0, j)),
        ],
        out_shape=[
            jax.ShapeDtypeStruct((nb, 2, HW), jnp.float32),
            jax.ShapeDtypeStruct((nb, 1, HW), jnp.float32),
            jax.ShapeDtypeStruct((nb, 2, HW), jnp.int32),
            jax.ShapeDtypeStruct((nb, 1, HW), jnp.int32),
        ],
        compiler_params=pltpu.CompilerParams(
            dimension_semantics=("arbitrary", "arbitrary")
        ),
    )(xf, conv_w, bb)


def _sc_scatter(xf, destf, D, b0, nb):
    B, C, HW = xf.shape
    NW = _NC * _NS
    assert NW % nb == 0
    WPB = NW // nb
    UNITS = C + 1
    q, r = divmod(UNITS, WPB)
    NIT = HW // _L
    NZD = D // _L
    CH = HW // 2
    NITC = CH // _L

    mesh = plsc.VectorSubcoreMesh(core_axis_name="c", subcore_axis_name="s")

    @functools.partial(
        pl.kernel,
        mesh=mesh,
        out_type=jax.ShapeDtypeStruct((nb * UNITS * D,), jnp.float32),
        scratch_types=[
            pltpu.VMEM((HW,), jnp.int32),
            pltpu.VMEM((CH,), jnp.float32),
            pltpu.VMEM((CH,), jnp.float32),
            pltpu.VMEM((D,), jnp.float32),
            pltpu.SemaphoreType.DMA,
            pltpu.SemaphoreType.DMA,
        ],
        compiler_params=pltpu.CompilerParams(needs_layout_passes=False),
    )
    def sc_kernel(x_hbm, dest_hbm, down_hbm, idx_v, xb0, xb1, acc, s0, s1):
        wid = lax.axis_index("s") * _NC + lax.axis_index("c")
        bl = wid // WPB
        b = b0 + bl
        wk = wid % WPB
        cnt = jnp.where(wk < r, q + 1, q)
        base = wk * q + jnp.minimum(wk, r)
        nreal = jnp.minimum(cnt, C - base)
        has_ones = (base + cnt) == UNITS

        pltpu.sync_copy(dest_hbm.at[pl.ds(bl * HW, HW)], idx_v)

        def zero_acc():
            @plsc.parallel_loop(0, NZD, unroll=8)
            def _z(i):
                acc[pl.ds(i * _L, _L)] = jnp.zeros((_L,), jnp.float32)

        xbufs = (xb0, xb1)
        sems = (s0, s1)

        def xoff(j, h):
            return (b * C + base + j) * HW + h * CH

        pltpu.async_copy(x_hbm.at[pl.ds(xoff(0, 0), CH)], xb0, s0)
        pltpu.async_copy(x_hbm.at[pl.ds(xoff(0, 1), CH)], xb1, s1)
        zero_acc()

        def chan_body(j, carry):
            for h in (0, 1):
                buf, sem = xbufs[h], sems[h]
                pltpu.make_async_copy(x_hbm.at[pl.ds(0, CH)], buf, sem).wait()

                @plsc.parallel_loop(0, NITC, unroll=16)
                def _scatter(i):
                    plsc.addupdate_scatter(
                        acc,
                        [idx_v[pl.ds(h * CH + i * _L, _L)]],
                        buf[pl.ds(i * _L, _L)],
                    )

                @pl.when(j + 1 < nreal)
                def _prefetch():
                    pltpu.async_copy(
                        x_hbm.at[pl.ds(xoff(j + 1, h), CH)], buf, sem
                    )

            pltpu.sync_copy(
                acc, down_hbm.at[pl.ds((bl * UNITS + base + j) * D, D)]
            )
            zero_acc()
            return carry

        lax.fori_loop(0, nreal, chan_body, 0)

        @pl.when(has_ones)
        def _count_channel():
            ones = jnp.ones((_L,), jnp.float32)

            @plsc.parallel_loop(0, NIT, unroll=16)
            def _ones_scatter(i):
                plsc.addupdate_scatter(acc, [idx_v[pl.ds(i * _L, _L)]], ones)

            inv = 1.0 / float(HW)

            @plsc.parallel_loop(0, NZD, unroll=8)
            def _scale(i):
                sl = pl.ds(i * _L, _L)
                acc[sl] = acc[sl] * inv

            pltpu.sync_copy(acc, down_hbm.at[pl.ds((bl * UNITS + C) * D, D)])

    return sc_kernel(xf.reshape(-1), destf.reshape(-1))


@jax.jit
def kernel(x, conv_w, conv_b):
    B, C, H, W = x.shape
    HW = H * W
    dh = int(round(H * DOWNSAMPLE))
    dw = int(round(W * DOWNSAMPLE))
    D = dh * dw

    xf = x.reshape(B, C, HW)
    bb = conv_b.reshape(3, 1)

    GB = 1
    offs, atts, dsts, downs = [], [], [], []
    for b0 in range(0, B, GB):
        off, att, dst, flat = _tc_phase(xf, conv_w, bb, H, W, dh, dw, b0, GB)
        offs.append(off)
        atts.append(att)
        dsts.append(dst)
        downs.append(_sc_scatter(xf, flat, D, b0, GB))

    down = jnp.concatenate(downs)
    off = jnp.concatenate(offs)
    att = jnp.concatenate(atts)
    dst = jnp.concatenate(dsts)

    return (
        down.reshape(B, C + 1, dh, dw),
        off.reshape(B, 2, H, W),
        att.reshape(B, 1, H, W),
        dst.reshape(B, 2, H, W),
    )


# speedup vs baseline: 1.8225x; 1.8225x over previous
import functools

import jax
import jax.numpy as jnp
from jax import lax
from jax.experimental import pallas as pl
from jax.experimental.pallas import tpu as pltpu
from jax.experimental.pallas import tpu_sc as plsc

EPS = 1e-05
DOWNSAMPLE = 0.5
_NC = 2
_NS = 16
_L = 16
_RH = 8


def _tc_body(x_ref, w_ref, b_ref, off_ref, att_ref, dst_ref, idx_ref,
             *, H, W, dh, dw):
    j = pl.program_id(1)
    xb = x_ref[0]

    ww = jax.lax.broadcasted_iota(jnp.int32, (1, W), 1).astype(jnp.float32)
    ww = ww * (1.0 / float(W))

    for rr in range(_RH):
        xrow = xb[:, rr, :]
        oa = (jnp.dot(w_ref[...], xrow, preferred_element_type=jnp.float32)
              + b_ref[...])

        h = (j * _RH + rr).astype(jnp.float32) * (1.0 / float(H))
        d0 = jnp.clip(h + oa[0:1], 0.0, 1.0 - EPS)
        d1 = jnp.clip(ww + oa[1:2], 0.0, 1.0 - EPS)
        dr = jnp.floor(d0 * dh).astype(jnp.int32)
        dc = jnp.floor(d1 * dw).astype(jnp.int32)

        off_ref[0, :, rr, :] = oa[0:2]
        att_ref[0, :, rr, :] = oa[2:3]
        dst_ref[0, :, rr, :] = jnp.concatenate([dr, dc], axis=0)
        idx_ref[0, :, rr, :] = dr * dw + dc


def _tc_phase(x, conv_w, bb, H, W, dh, dw):
    B, C, _, _ = x.shape
    assert H % _RH == 0
    NJ = H // _RH
    return pl.pallas_call(
        functools.partial(_tc_body, H=H, W=W, dh=dh, dw=dw),
        grid=(B, NJ),
        in_specs=[
            pl.BlockSpec((1, C, _RH, W), lambda b, j: (b, 0, j, 0)),
            pl.BlockSpec((3, C), lambda b, j: (0, 0)),
            pl.BlockSpec((3, 1), lambda b, j: (0, 0)),
        ],
        out_specs=[
            pl.BlockSpec((1, 2, _RH, W), lambda b, j: (b, 0, j, 0)),
            pl.BlockSpec((1, 1, _RH, W), lambda b, j: (b, 0, j, 0)),
            pl.BlockSpec((1, 2, _RH, W), lambda b, j: (b, 0, j, 0)),
            pl.BlockSpec((1, 1, _RH, W), lambda b, j: (b, 0, j, 0)),
        ],
        out_shape=[
            jax.ShapeDtypeStruct((B, 2, H, W), jnp.float32),
            jax.ShapeDtypeStruct((B, 1, H, W), jnp.float32),
            jax.ShapeDtypeStruct((B, 2, H, W), jnp.int32),
            jax.ShapeDtypeStruct((B, 1, H, W), jnp.int32),
        ],
        compiler_params=pltpu.CompilerParams(
            dimension_semantics=("arbitrary", "arbitrary")
        ),
    )(x, conv_w, bb)


def _sc_scatter(x, idx4, dh, dw):
    B, C, H, W = x.shape
    D = dh * dw
    NW = _NC * _NS
    assert NW % B == 0 and W % _L == 0
    WPB = NW // B
    UNITS = C + 1
    q, r = divmod(UNITS, WPB)
    NZD = D // _L
    RC = H // 4
    NV = W // _L

    mesh = plsc.VectorSubcoreMesh(core_axis_name="c", subcore_axis_name="s")

    @functools.partial(
        pl.kernel,
        mesh=mesh,
        out_type=jax.ShapeDtypeStruct((B, UNITS, dh, dw), jnp.float32),
        scratch_types=[
            pltpu.VMEM((1, 1, H, W), jnp.int32),
            pltpu.VMEM((1, 1, RC, W), jnp.float32),
            pltpu.VMEM((1, 1, RC, W), jnp.float32),
            pltpu.VMEM((D,), jnp.float32),
            pltpu.VMEM((1, 1, dh, dw), jnp.float32),
            pltpu.SemaphoreType.DMA,
            pltpu.SemaphoreType.DMA,
        ],
        compiler_params=pltpu.CompilerParams(needs_layout_passes=False),
    )
    def sc_kernel(x_hbm, idx_hbm, down_hbm, idx_v, xb0, xb1, acc, acc4, s0, s1):
        NVD = dw // _L

        def stage_acc():
            @plsc.parallel_loop(0, dh, unroll=2)
            def _cp(rw):
                for v in range(NVD):
                    acc4[0, 0, rw, pl.ds(v * _L, _L)] = (
                        acc[pl.ds(rw * dw + v * _L, _L)]
                    )
        wid = lax.axis_index("s") * _NC + lax.axis_index("c")
        b = wid // WPB
        wk = wid % WPB
        cnt = jnp.where(wk < r, q + 1, q)
        base = wk * q + jnp.minimum(wk, r)
        nreal = jnp.minimum(cnt, C - base)
        has_ones = (base + cnt) == UNITS

        pltpu.sync_copy(idx_hbm.at[pl.ds(b, 1)], idx_v)

        def zero_acc():
            @plsc.parallel_loop(0, NZD, unroll=8)
            def _z(i):
                acc[pl.ds(i * _L, _L)] = jnp.zeros((_L,), jnp.float32)

        xbufs = (xb0, xb1)
        sems = (s0, s1)

        def fetch(j, h, sem, buf):
            pltpu.async_copy(
                x_hbm.at[pl.ds(b, 1), pl.ds(base + j, 1), pl.ds(h * RC, RC)],
                buf, sem,
            )

        def wait(sem, buf):
            pltpu.make_async_copy(
                x_hbm.at[pl.ds(0, 1), pl.ds(0, 1), pl.ds(0, RC)], buf, sem
            ).wait()

        fetch(0, 0, s0, xb0)
        fetch(0, 1, s1, xb1)
        zero_acc()

        def chan_body(j, carry):
            for h in range(4):
                buf, sem = xbufs[h % 2], sems[h % 2]
                wait(sem, buf)
                roff = h * RC

                @plsc.parallel_loop(0, RC, unroll=2)
                def _rows(rw):
                    for v in range(NV):
                        sl = pl.ds(v * _L, _L)
                        plsc.addupdate_scatter(
                            acc, [idx_v[0, 0, roff + rw, sl]],
                            buf[0, 0, rw, sl],
                        )

                if h < 2:
                    fetch(j, h + 2, sem, buf)
                else:
                    @pl.when(j + 1 < nreal)
                    def _prefetch():
                        fetch(j + 1, h - 2, sem, buf)

            stage_acc()
            pltpu.sync_copy(
                acc4, down_hbm.at[pl.ds(b, 1), pl.ds(base + j, 1)]
            )
            zero_acc()
            return carry

        lax.fori_loop(0, nreal, chan_body, 0)

        @pl.when(has_ones)
        def _count_channel():
            ones = jnp.ones((_L,), jnp.float32)

            @plsc.parallel_loop(0, H, unroll=2)
            def _rows(rw):
                for v in range(NV):
                    plsc.addupdate_scatter(
                        acc, [idx_v[0, 0, rw, pl.ds(v * _L, _L)]], ones
                    )

            inv = 1.0 / float(H * W)

            @plsc.parallel_loop(0, NZD, unroll=8)
            def _scale(i):
                sl = pl.ds(i * _L, _L)
                acc[sl] = acc[sl] * inv

            stage_acc()
            pltpu.sync_copy(
                acc4, down_hbm.at[pl.ds(b, 1), pl.ds(C, 1)]
            )

    return sc_kernel(x, idx4)


@jax.jit
def kernel(x, conv_w, conv_b):
    B, C, H, W = x.shape
    dh = int(round(H * DOWNSAMPLE))
    dw = int(round(W * DOWNSAMPLE))

    bb = conv_b.reshape(3, 1)
    off, att, dst, idx4 = _tc_phase(x, conv_w, bb, H, W, dh, dw)
    down = _sc_scatter(x, idx4, dh, dw)
    return (down, off, att, dst)


# speedup vs baseline: 2.0011x; 1.0980x over previous
import functools

import jax
import jax.numpy as jnp
from jax import lax
from jax.experimental import pallas as pl
from jax.experimental.pallas import tpu as pltpu
from jax.experimental.pallas import tpu_sc as plsc

EPS = 1e-05
DOWNSAMPLE = 0.5
_NC = 2
_NS = 16
_L = 16
_RH = 16


def _tc_body(x_ref, w_ref, b_ref, off_ref, att_ref, dst_ref, idx_ref,
             *, H, W, dh, dw):
    j = pl.program_id(1)
    xb = x_ref[0]

    ww = jax.lax.broadcasted_iota(jnp.int32, (1, W), 1).astype(jnp.float32)
    ww = ww * (1.0 / float(W))

    for rr in range(_RH):
        xrow = xb[:, rr, :]
        oa = (jnp.dot(w_ref[...], xrow, preferred_element_type=jnp.float32)
              + b_ref[...])

        h = (j * _RH + rr).astype(jnp.float32) * (1.0 / float(H))
        d0 = jnp.clip(h + oa[0:1], 0.0, 1.0 - EPS)
        d1 = jnp.clip(ww + oa[1:2], 0.0, 1.0 - EPS)
        dr = jnp.floor(d0 * dh).astype(jnp.int32)
        dc = jnp.floor(d1 * dw).astype(jnp.int32)

        off_ref[0, :, rr, :] = oa[0:2]
        att_ref[0, :, rr, :] = oa[2:3]
        dst_ref[0, :, rr, :] = jnp.concatenate([dr, dc], axis=0)
        idx_ref[0, :, rr, :] = dr * dw + dc


def _tc_phase(x, conv_w, bb, H, W, dh, dw):
    B, C, _, _ = x.shape
    assert H % _RH == 0
    NJ = H // _RH
    return pl.pallas_call(
        functools.partial(_tc_body, H=H, W=W, dh=dh, dw=dw),
        grid=(B, NJ),
        in_specs=[
            pl.BlockSpec((1, C, _RH, W), lambda b, j: (b, 0, j, 0)),
            pl.BlockSpec((3, C), lambda b, j: (0, 0)),
            pl.BlockSpec((3, 1), lambda b, j: (0, 0)),
        ],
        out_specs=[
            pl.BlockSpec((1, 2, _RH, W), lambda b, j: (b, 0, j, 0)),
            pl.BlockSpec((1, 1, _RH, W), lambda b, j: (b, 0, j, 0)),
            pl.BlockSpec((1, 2, _RH, W), lambda b, j: (b, 0, j, 0)),
            pl.BlockSpec((1, 1, _RH, W), lambda b, j: (b, 0, j, 0)),
        ],
        out_shape=[
            jax.ShapeDtypeStruct((B, 2, H, W), jnp.float32),
            jax.ShapeDtypeStruct((B, 1, H, W), jnp.float32),
            jax.ShapeDtypeStruct((B, 2, H, W), jnp.int32),
            jax.ShapeDtypeStruct((B, 1, H, W), jnp.int32),
        ],
        compiler_params=pltpu.CompilerParams(
            dimension_semantics=("arbitrary", "arbitrary")
        ),
    )(x, conv_w, bb)


def _sc_scatter(x, idx4, dh, dw):
    B, C, H, W = x.shape
    D = dh * dw
    NW = _NC * _NS
    assert NW % B == 0 and W % _L == 0
    WPB = NW // B
    UNITS = C + 1
    q, r = divmod(UNITS, WPB)
    NZD = D // _L
    RC = H // 4
    NV = W // _L

    mesh = plsc.VectorSubcoreMesh(core_axis_name="c", subcore_axis_name="s")

    @functools.partial(
        pl.kernel,
        mesh=mesh,
        out_type=jax.ShapeDtypeStruct((B, UNITS, dh, dw), jnp.float32),
        scratch_types=[
            pltpu.VMEM((1, 1, H, W), jnp.int32),
            pltpu.VMEM((1, 1, RC, W), jnp.float32),
            pltpu.VMEM((1, 1, RC, W), jnp.float32),
            pltpu.VMEM((D,), jnp.float32),
            pltpu.VMEM((1, 1, dh, dw), jnp.float32),
            pltpu.SemaphoreType.DMA,
            pltpu.SemaphoreType.DMA,
        ],
        compiler_params=pltpu.CompilerParams(needs_layout_passes=False),
    )
    def sc_kernel(x_hbm, idx_hbm, down_hbm, idx_v, xb0, xb1, acc, acc4, s0, s1):
        NVD = dw // _L

        def stage_acc():
            @plsc.parallel_loop(0, dh, unroll=2)
            def _cp(rw):
                for v in range(NVD):
                    acc4[0, 0, rw, pl.ds(v * _L, _L)] = (
                        acc[pl.ds(rw * dw + v * _L, _L)]
                    )
        wid = lax.axis_index("s") * _NC + lax.axis_index("c")
        b = wid // WPB
        wk = wid % WPB
        cnt = jnp.where(wk < r, q + 1, q)
        base = wk * q + jnp.minimum(wk, r)
        nreal = jnp.minimum(cnt, C - base)
        has_ones = (base + cnt) == UNITS

        pltpu.sync_copy(idx_hbm.at[pl.ds(b, 1)], idx_v)

        def zero_acc():
            @plsc.parallel_loop(0, NZD, unroll=8)
            def _z(i):
                acc[pl.ds(i * _L, _L)] = jnp.zeros((_L,), jnp.float32)

        xbufs = (xb0, xb1)
        sems = (s0, s1)

        def fetch(j, h, sem, buf):
            pltpu.async_copy(
                x_hbm.at[pl.ds(b, 1), pl.ds(base + j, 1), pl.ds(h * RC, RC)],
                buf, sem,
            )

        def wait(sem, buf):
            pltpu.make_async_copy(
                x_hbm.at[pl.ds(0, 1), pl.ds(0, 1), pl.ds(0, RC)], buf, sem
            ).wait()

        fetch(0, 0, s0, xb0)
        fetch(0, 1, s1, xb1)
        zero_acc()

        def chan_body(j, carry):
            for h in range(4):
                buf, sem = xbufs[h % 2], sems[h % 2]
                wait(sem, buf)
                roff = h * RC

                @plsc.parallel_loop(0, RC, unroll=4)
                def _rows(rw):
                    for v in range(NV):
                        sl = pl.ds(v * _L, _L)
                        plsc.addupdate_scatter(
                            acc, [idx_v[0, 0, roff + rw, sl]],
                            buf[0, 0, rw, sl],
                        )

                if h < 2:
                    fetch(j, h + 2, sem, buf)
                else:
                    @pl.when(j + 1 < nreal)
                    def _prefetch():
                        fetch(j + 1, h - 2, sem, buf)

            stage_acc()
            pltpu.sync_copy(
                acc4, down_hbm.at[pl.ds(b, 1), pl.ds(base + j, 1)]
            )
            zero_acc()
            return carry

        lax.fori_loop(0, nreal, chan_body, 0)

        @pl.when(has_ones)
        def _count_channel():
            ones = jnp.ones((_L,), jnp.float32)

            @plsc.parallel_loop(0, H, unroll=2)
            def _rows(rw):
                for v in range(NV):
                    plsc.addupdate_scatter(
                        acc, [idx_v[0, 0, rw, pl.ds(v * _L, _L)]], ones
                    )

            inv = 1.0 / float(H * W)

            @plsc.parallel_loop(0, NZD, unroll=8)
            def _scale(i):
                sl = pl.ds(i * _L, _L)
                acc[sl] = acc[sl] * inv

            stage_acc()
            pltpu.sync_copy(
                acc4, down_hbm.at[pl.ds(b, 1), pl.ds(C, 1)]
            )

    return sc_kernel(x, idx4)


@jax.jit
def kernel(x, conv_w, conv_b):
    B, C, H, W = x.shape
    dh = int(round(H * DOWNSAMPLE))
    dw = int(round(W * DOWNSAMPLE))

    bb = conv_b.reshape(3, 1)
    off, att, dst, idx4 = _tc_phase(x, conv_w, bb, H, W, dh, dw)
    down = _sc_scatter(x, idx4, dh, dw)
    return (down, off, att, dst)


# speedup vs baseline: 2.1271x; 1.0630x over previous
import functools

import jax
import jax.numpy as jnp
from jax import lax
from jax.experimental import pallas as pl
from jax.experimental.pallas import tpu as pltpu
from jax.experimental.pallas import tpu_sc as plsc

EPS = 1e-05
DOWNSAMPLE = 0.5
_NC = 2
_NS = 16
_L = 16
_RH = 32


def _tc_body(x_ref, w_ref, b_ref, off_ref, att_ref, dst_ref, idx_ref,
             *, H, W, dh, dw):
    j = pl.program_id(1)
    xb = x_ref[0]

    ww = jax.lax.broadcasted_iota(jnp.int32, (1, W), 1).astype(jnp.float32)
    ww = ww * (1.0 / float(W))

    for rr in range(_RH):
        xrow = xb[:, rr, :]
        oa = (jnp.dot(w_ref[...], xrow, preferred_element_type=jnp.float32)
              + b_ref[...])

        h = (j * _RH + rr).astype(jnp.float32) * (1.0 / float(H))
        d0 = jnp.clip(h + oa[0:1], 0.0, 1.0 - EPS)
        d1 = jnp.clip(ww + oa[1:2], 0.0, 1.0 - EPS)
        dr = jnp.floor(d0 * dh).astype(jnp.int32)
        dc = jnp.floor(d1 * dw).astype(jnp.int32)

        off_ref[0, :, rr, :] = oa[0:2]
        att_ref[0, :, rr, :] = oa[2:3]
        dst_ref[0, :, rr, :] = jnp.concatenate([dr, dc], axis=0)
        idx_ref[0, :, rr, :] = dr * dw + dc


def _tc_phase(x, conv_w, bb, H, W, dh, dw):
    B, C, _, _ = x.shape
    assert H % _RH == 0
    NJ = H // _RH
    return pl.pallas_call(
        functools.partial(_tc_body, H=H, W=W, dh=dh, dw=dw),
        grid=(B, NJ),
        in_specs=[
            pl.BlockSpec((1, C, _RH, W), lambda b, j: (b, 0, j, 0)),
            pl.BlockSpec((3, C), lambda b, j: (0, 0)),
            pl.BlockSpec((3, 1), lambda b, j: (0, 0)),
        ],
        out_specs=[
            pl.BlockSpec((1, 2, _RH, W), lambda b, j: (b, 0, j, 0)),
            pl.BlockSpec((1, 1, _RH, W), lambda b, j: (b, 0, j, 0)),
            pl.BlockSpec((1, 2, _RH, W), lambda b, j: (b, 0, j, 0)),
            pl.BlockSpec((1, 1, _RH, W), lambda b, j: (b, 0, j, 0)),
        ],
        out_shape=[
            jax.ShapeDtypeStruct((B, 2, H, W), jnp.float32),
            jax.ShapeDtypeStruct((B, 1, H, W), jnp.float32),
            jax.ShapeDtypeStruct((B, 2, H, W), jnp.int32),
            jax.ShapeDtypeStruct((B, 1, H, W), jnp.int32),
        ],
        compiler_params=pltpu.CompilerParams(
            dimension_semantics=("arbitrary", "arbitrary")
        ),
    )(x, conv_w, bb)


def _sc_scatter(x, idx4, dh, dw):
    B, C, H, W = x.shape
    D = dh * dw
    NW = _NC * _NS
    assert NW % B == 0 and W % _L == 0
    WPB = NW // B
    UNITS = C + 1
    q, r = divmod(UNITS, WPB)
    NZD = D // _L
    RC = H // 4
    NV = W // _L

    mesh = plsc.VectorSubcoreMesh(core_axis_name="c", subcore_axis_name="s")

    @functools.partial(
        pl.kernel,
        mesh=mesh,
        out_type=jax.ShapeDtypeStruct((B, UNITS, dh, dw), jnp.float32),
        scratch_types=[
            pltpu.VMEM((1, 1, H, W), jnp.int32),
            pltpu.VMEM((1, 1, RC, W), jnp.float32),
            pltpu.VMEM((1, 1, RC, W), jnp.float32),
            pltpu.VMEM((D,), jnp.float32),
            pltpu.VMEM((1, 1, dh, dw), jnp.float32),
            pltpu.SemaphoreType.DMA,
            pltpu.SemaphoreType.DMA,
        ],
        compiler_params=pltpu.CompilerParams(needs_layout_passes=False),
    )
    def sc_kernel(x_hbm, idx_hbm, down_hbm, idx_v, xb0, xb1, acc, acc4, s0, s1):
        NVD = dw // _L

        def stage_acc():
            @plsc.parallel_loop(0, dh, unroll=2)
            def _cp(rw):
                for v in range(NVD):
                    acc4[0, 0, rw, pl.ds(v * _L, _L)] = (
                        acc[pl.ds(rw * dw + v * _L, _L)]
                    )
        wid = lax.axis_index("s") * _NC + lax.axis_index("c")
        b = wid // WPB
        wk = wid % WPB
        cnt = jnp.where(wk < r, q + 1, q)
        base = wk * q + jnp.minimum(wk, r)
        nreal = jnp.minimum(cnt, C - base)
        has_ones = (base + cnt) == UNITS

        pltpu.sync_copy(idx_hbm.at[pl.ds(b, 1)], idx_v)

        def zero_acc():
            @plsc.parallel_loop(0, NZD, unroll=8)
            def _z(i):
                acc[pl.ds(i * _L, _L)] = jnp.zeros((_L,), jnp.float32)

        xbufs = (xb0, xb1)
        sems = (s0, s1)

        def fetch(j, h, sem, buf):
            pltpu.async_copy(
                x_hbm.at[pl.ds(b, 1), pl.ds(base + j, 1), pl.ds(h * RC, RC)],
                buf, sem,
            )

        def wait(sem, buf):
            pltpu.make_async_copy(
                x_hbm.at[pl.ds(0, 1), pl.ds(0, 1), pl.ds(0, RC)], buf, sem
            ).wait()

        fetch(0, 0, s0, xb0)
        fetch(0, 1, s1, xb1)
        zero_acc()

        def chan_body(j, carry):
            for h in range(4):
                buf, sem = xbufs[h % 2], sems[h % 2]
                wait(sem, buf)
                roff = h * RC

                @plsc.parallel_loop(0, RC, unroll=4)
                def _rows(rw):
                    for v in range(NV):
                        sl = pl.ds(v * _L, _L)
                        plsc.addupdate_scatter(
                            acc, [idx_v[0, 0, roff + rw, sl]],
                            buf[0, 0, rw, sl],
                        )

                if h < 2:
                    fetch(j, h + 2, sem, buf)
                else:
                    @pl.when(j + 1 < nreal)
                    def _prefetch():
                        fetch(j + 1, h - 2, sem, buf)

            stage_acc()
            pltpu.sync_copy(
                acc4, down_hbm.at[pl.ds(b, 1), pl.ds(base + j, 1)]
            )
            zero_acc()
            return carry

        lax.fori_loop(0, nreal, chan_body, 0)

        @pl.when(has_ones)
        def _count_channel():
            ones = jnp.ones((_L,), jnp.float32)

            @plsc.parallel_loop(0, H, unroll=2)
            def _rows(rw):
                for v in range(NV):
                    plsc.addupdate_scatter(
                        acc, [idx_v[0, 0, rw, pl.ds(v * _L, _L)]], ones
                    )

            inv = 1.0 / float(H * W)

            @plsc.parallel_loop(0, NZD, unroll=8)
            def _scale(i):
                sl = pl.ds(i * _L, _L)
                acc[sl] = acc[sl] * inv

            stage_acc()
            pltpu.sync_copy(
                acc4, down_hbm.at[pl.ds(b, 1), pl.ds(C, 1)]
            )

    return sc_kernel(x, idx4)


@jax.jit
def kernel(x, conv_w, conv_b):
    B, C, H, W = x.shape
    dh = int(round(H * DOWNSAMPLE))
    dw = int(round(W * DOWNSAMPLE))

    bb = conv_b.reshape(3, 1)
    off, att, dst, idx4 = _tc_phase(x, conv_w, bb, H, W, dh, dw)
    down = _sc_scatter(x, idx4, dh, dw)
    return (down, off, att, dst)
